# baseline (speedup 1.0000x reference)
import functools
import jax
import jax.numpy as jnp
from jax import lax
from jax.experimental import pallas as pl
from jax.experimental.pallas import tpu as pltpu
from jax.experimental.pallas import tpu_sc as plsc

N_CORES = 2
N_SUBCORES = 16
N_TILES = N_CORES * N_SUBCORES
CHUNK = 125


def _make_seg_sum(n_rows: int, n_acc: int, d: int, chunks_per_tile: int,
                  dtype=jnp.bfloat16):
  rows_per_sub = n_acc // N_SUBCORES
  tab_per_sub = n_rows // N_SUBCORES
  mesh = plsc.VectorSubcoreMesh(core_axis_name="c", subcore_axis_name="s")

  def body(rows_hbm, src_hbm, dst_hbm, zeros_hbm, out_hbm,
           src_v, dst_v, buf0_v, buf1_v, tab_sh, acc_sh,
           gsem0, gsem1, ssem0, ssem1):
    cid = lax.axis_index("c")
    sid = lax.axis_index("s")
    wid = cid * N_SUBCORES + sid
    pltpu.sync_copy(rows_hbm.at[pl.ds(sid * tab_per_sub, tab_per_sub)],
                    tab_sh.at[pl.ds(sid * tab_per_sub, tab_per_sub)])
    pltpu.sync_copy(zeros_hbm.at[pl.ds(sid * rows_per_sub, rows_per_sub)],
                    acc_sh.at[pl.ds(sid * rows_per_sub, rows_per_sub)])
    pltpu.sync_copy(src_hbm.at[pl.ds(wid * chunks_per_tile, chunks_per_tile)],
                    src_v)
    pltpu.sync_copy(dst_hbm.at[pl.ds(wid * chunks_per_tile, chunks_per_tile)],
                    dst_v)
    plsc.subcore_barrier()

    g0 = pltpu.async_copy(tab_sh.at[src_v.at[0]], buf0_v, gsem0)

    @pl.loop(0, chunks_per_tile, step=2)
    def _chunk(c):
      g0.wait()
      s0 = pltpu.async_copy(buf0_v, acc_sh.at[dst_v.at[c]], ssem0, add=True)
      g1 = pltpu.async_copy(tab_sh.at[src_v.at[c + 1]], buf1_v, gsem1)
      s0.wait()
      g1.wait()
      s1 = pltpu.async_copy(buf1_v, acc_sh.at[dst_v.at[c + 1]], ssem1,
                            add=True)

      @pl.when(c + 2 < chunks_per_tile)
      def _():
        pltpu.async_copy(tab_sh.at[src_v.at[c + 2]], buf0_v, gsem0)

      s1.wait()

    plsc.subcore_barrier()
    pltpu.sync_copy(acc_sh.at[pl.ds(sid * rows_per_sub, rows_per_sub)],
                    out_hbm.at[cid, pl.ds(sid * rows_per_sub, rows_per_sub)])

  return pl.kernel(
      body,
      out_type=jax.ShapeDtypeStruct((N_CORES, n_acc, d), dtype),
      mesh=mesh,
      compiler_params=pltpu.CompilerParams(use_tc_tiling_on_sc=False),
      scratch_types=[
          pltpu.VMEM((chunks_per_tile, CHUNK), jnp.int32),
          pltpu.VMEM((chunks_per_tile, CHUNK), jnp.int32),
          pltpu.VMEM((CHUNK, d), dtype),
          pltpu.VMEM((CHUNK, d), dtype),
          pltpu.VMEM_SHARED((n_rows, d), dtype),
          pltpu.VMEM_SHARED((n_acc, d), dtype),
          pltpu.SemaphoreType.DMA,
          pltpu.SemaphoreType.DMA,
          pltpu.SemaphoreType.DMA,
          pltpu.SemaphoreType.DMA,
      ],
  )


_DOT_FAST = functools.partial(jnp.dot, preferred_element_type=jnp.float32,
                              precision=lax.Precision.DEFAULT)
_DOT = functools.partial(jnp.dot, preferred_element_type=jnp.float32,
                         precision=lax.Precision.HIGHEST)


def _p_body(x_ref, wl_ref, pext_ref):
  x = x_ref[...]
  p = _DOT_FAST(x, wl_ref[...])
  ones = jnp.ones((x.shape[0], 32), jnp.float32)
  pext_ref[...] = jnp.concatenate([p, ones], axis=1).astype(jnp.bfloat16)


def _base_body(x_ref, wre_ref, wro_ref, be_ref, bo_ref, bse_ref, bso_ref):
  x = x_ref[...]
  bse_ref[...] = _DOT(x, wre_ref[...]) + be_ref[...]
  bso_ref[...] = _DOT(x, wro_ref[...]) + bo_ref[...]


def _unpack_even(a):
  return jax.lax.bitcast_convert_type(
      jax.lax.shift_left(a, 16), jnp.float32)


def _unpack_odd(a):
  return jax.lax.bitcast_convert_type(
      jax.lax.bitwise_and(a, jnp.int32(-65536)), jnp.float32)


def _mid_body(part_ref, bse_ref, bso_ref, ge_ref, go_ref, bte_ref, bto_ref,
              wle_ref, wlo_ref, q_ref, he_ref, ho_ref, cnt_ref):
  half = bse_ref.shape[1]

  def _bn_relu(agg, cnt, base, g, bt):
    h = agg / cnt + base
    mu = jnp.mean(h, axis=0, keepdims=True)
    var = jnp.mean((h - mu) ** 2, axis=0, keepdims=True)
    return jnp.maximum((h - mu) / jnp.sqrt(var + 1e-5) * g + bt, 0.0)

  lo0 = _unpack_even(part_ref[0])
  lo1 = _unpack_even(part_ref[1])
  cnt = jnp.maximum(lo0[:, half:half + 1] + lo1[:, half:half + 1], 1.0)
  he = _bn_relu(lo0[:, :half] + lo1[:, :half], cnt, bse_ref[...],
                ge_ref[...], bte_ref[...])
  ho = _bn_relu(_unpack_odd(part_ref[0])[:, :half] +
                _unpack_odd(part_ref[1])[:, :half], cnt, bso_ref[...],
                go_ref[...], bto_ref[...])
  q_ref[...] = (_DOT_FAST(he, wle_ref[...]) +
                _DOT_FAST(ho, wlo_ref[...])).astype(jnp.bfloat16)
  he_ref[...] = he
  ho_ref[...] = ho
  cnt_ref[...] = cnt


def _s_body(he_ref, ho_ref, wre_ref, wro_ref, b2_ref, s_ref):
  s_ref[...] = (_DOT(he_ref[...], wre_ref[...]) +
                _DOT(ho_ref[...], wro_ref[...]) + b2_ref[...])


def _out_body(part_ref, s_ref, cnt_ref, out_ref):
  aggq = (part_ref[0].astype(jnp.float32) +
          part_ref[1].astype(jnp.float32))
  z = aggq / cnt_ref[...] + s_ref[...]
  norm = jnp.sqrt(jnp.sum(z * z, axis=1, keepdims=True))
  out_ref[...] = z / jnp.maximum(norm, 1e-12)


@jax.jit
def kernel(x, edge_index, W1l, b1l, W1r, b1r, gamma, beta, W2l, b2l, W2r, b2r):
  n, in_dim = x.shape
  hid = W1l.shape[1]
  out_dim = W2l.shape[1]
  n_edges = edge_index.shape[1]

  e_pad = -(-n_edges // (2 * N_TILES * CHUNK)) * (2 * N_TILES * CHUNK)
  n_acc = -(-(n + (1 if e_pad != n_edges else 0)) //
            (8 * N_SUBCORES)) * (8 * N_SUBCORES)

  src = edge_index[0].astype(jnp.int32)
  dst = edge_index[1].astype(jnp.int32)
  if e_pad != n_edges:
    src = jnp.concatenate([src, jnp.zeros((e_pad - n_edges,), jnp.int32)])
    dst = jnp.concatenate(
        [dst, jnp.full((e_pad - n_edges,), n_acc - 1, jnp.int32)])
  total_chunks = e_pad // CHUNK
  chunks_per_tile = total_chunks // N_TILES
  src2d = src.reshape(total_chunks, CHUNK)
  dst2d = dst.reshape(total_chunks, CHUNK)

  dp1 = hid + 32
  zeros1 = jnp.zeros((n_acc, dp1), jnp.bfloat16)
  zeros2 = jnp.zeros((n_acc, out_dim), jnp.bfloat16)

  blk = 1000
  grid = n // blk
  b1 = (b1l + b1r).reshape(1, hid)
  pext_pad = pl.pallas_call(
      _p_body,
      grid=(grid,),
      in_specs=[
          pl.BlockSpec((blk, in_dim), lambda i: (i, 0)),
          pl.BlockSpec((in_dim, hid), lambda i: (0, 0)),
      ],
      out_specs=pl.BlockSpec((blk, dp1), lambda i: (i, 0)),
      out_shape=jax.ShapeDtypeStruct((n, dp1), jnp.bfloat16),
  )(x, W1l)

  part1 = _make_seg_sum(n, n_acc, dp1, chunks_per_tile)(
      pext_pad, src2d, dst2d, zeros1)

  half = hid // 2
  base_ev, base_od = pl.pallas_call(
      _base_body,
      grid=(grid,),
      in_specs=[
          pl.BlockSpec((blk, in_dim), lambda i: (i, 0)),
          pl.BlockSpec((in_dim, half), lambda i: (0, 0)),
          pl.BlockSpec((in_dim, half), lambda i: (0, 0)),
          pl.BlockSpec((1, half), lambda i: (0, 0)),
          pl.BlockSpec((1, half), lambda i: (0, 0)),
      ],
      out_specs=[
          pl.BlockSpec((blk, half), lambda i: (i, 0)),
          pl.BlockSpec((blk, half), lambda i: (i, 0)),
      ],
      out_shape=[
          jax.ShapeDtypeStruct((n, half), jnp.float32),
          jax.ShapeDtypeStruct((n, half), jnp.float32),
      ],
  )(x, W1r[:, 0::2], W1r[:, 1::2], b1[:, 0::2], b1[:, 1::2])

  part1_i32 = jax.lax.bitcast_convert_type(
      part1.reshape(N_CORES, n_acc, dp1 // 2, 2), jnp.int32)
  b2 = (b2l + b2r).reshape(1, out_dim)
  gm = gamma.reshape(1, hid)
  bt = beta.reshape(1, hid)
  q, he, ho, cnt = pl.pallas_call(
      _mid_body,
      compiler_params=pltpu.CompilerParams(
          vmem_limit_bytes=100 * 1024 * 1024),
      grid=(1,),
      in_specs=[
          pl.BlockSpec((2, n, dp1 // 2), lambda i: (0, 0, 0)),
          pl.BlockSpec((n, half), lambda i: (0, 0)),
          pl.BlockSpec((n, half), lambda i: (0, 0)),
          pl.BlockSpec((1, half), lambda i: (0, 0)),
          pl.BlockSpec((1, half), lambda i: (0, 0)),
          pl.BlockSpec((1, half), lambda i: (0, 0)),
          pl.BlockSpec((1, half), lambda i: (0, 0)),
          pl.BlockSpec((half, out_dim), lambda i: (0, 0)),
          pl.BlockSpec((half, out_dim), lambda i: (0, 0)),
      ],
      out_specs=[
          pl.BlockSpec((n, out_dim), lambda i: (0, 0)),
          pl.BlockSpec((n, half), lambda i: (0, 0)),
          pl.BlockSpec((n, half), lambda i: (0, 0)),
          pl.BlockSpec((n, 1), lambda i: (0, 0)),
      ],
      out_shape=[
          jax.ShapeDtypeStruct((n, out_dim), jnp.bfloat16),
          jax.ShapeDtypeStruct((n, half), jnp.float32),
          jax.ShapeDtypeStruct((n, half), jnp.float32),
          jax.ShapeDtypeStruct((n, 1), jnp.float32),
      ],
  )(part1_i32, base_ev, base_od, gm[:, 0::2], gm[:, 1::2],
    bt[:, 0::2], bt[:, 1::2], W2l[0::2], W2l[1::2])

  part2 = _make_seg_sum(n, n_acc, out_dim, chunks_per_tile)(
      q, src2d, dst2d, zeros2)

  s = pl.pallas_call(
      _s_body,
      grid=(grid,),
      in_specs=[
          pl.BlockSpec((blk, half), lambda i: (i, 0)),
          pl.BlockSpec((blk, half), lambda i: (i, 0)),
          pl.BlockSpec((half, out_dim), lambda i: (0, 0)),
          pl.BlockSpec((half, out_dim), lambda i: (0, 0)),
          pl.BlockSpec((1, out_dim), lambda i: (0, 0)),
      ],
      out_specs=pl.BlockSpec((blk, out_dim), lambda i: (i, 0)),
      out_shape=jax.ShapeDtypeStruct((n, out_dim), jnp.float32),
  )(he, ho, W2r[0::2], W2r[1::2], b2)

  z = pl.pallas_call(
      _out_body,
      grid=(grid,),
      in_specs=[
          pl.BlockSpec((2, blk, out_dim), lambda i: (0, i, 0)),
          pl.BlockSpec((blk, out_dim), lambda i: (i, 0)),
          pl.BlockSpec((blk, 1), lambda i: (i, 0)),
      ],
      out_specs=pl.BlockSpec((blk, out_dim), lambda i: (i, 0)),
      out_shape=jax.ShapeDtypeStruct((n, out_dim), jnp.float32),
  )(part2, s, cnt)
  return z


# speedup vs baseline: 1.4813x; 1.4813x over previous
import functools
import jax
import jax.numpy as jnp
from jax import lax
from jax.experimental import pallas as pl
from jax.experimental.pallas import tpu as pltpu
from jax.experimental.pallas import tpu_sc as plsc

N_CORES = 2
N_SUBCORES = 16
N_TILES = N_CORES * N_SUBCORES
CHUNK = 125


def _make_seg_sum(n_rows: int, n_acc: int, d: int, chunks_per_tile: int,
                  dtype=jnp.bfloat16):
  rows_per_sub = n_acc // N_SUBCORES
  tab_per_sub = n_rows // N_SUBCORES
  mesh = plsc.VectorSubcoreMesh(core_axis_name="c", subcore_axis_name="s")

  def body(rows_hbm, src_hbm, dst_hbm, zeros_hbm, out_hbm,
           src_v, dst_v, buf0_v, buf1_v, tab_sh, acc_sh,
           gsem0, gsem1, ssem0, ssem1):
    cid = lax.axis_index("c")
    sid = lax.axis_index("s")
    wid = cid * N_SUBCORES + sid
    pltpu.sync_copy(rows_hbm.at[pl.ds(sid * tab_per_sub, tab_per_sub)],
                    tab_sh.at[pl.ds(sid * tab_per_sub, tab_per_sub)])
    pltpu.sync_copy(zeros_hbm.at[pl.ds(sid * rows_per_sub, rows_per_sub)],
                    acc_sh.at[pl.ds(sid * rows_per_sub, rows_per_sub)])
    pltpu.sync_copy(src_hbm.at[pl.ds(wid * chunks_per_tile, chunks_per_tile)],
                    src_v)
    pltpu.sync_copy(dst_hbm.at[pl.ds(wid * chunks_per_tile, chunks_per_tile)],
                    dst_v)
    plsc.subcore_barrier()

    g0 = pltpu.async_copy(tab_sh.at[src_v.at[0]], buf0_v, gsem0)

    @pl.loop(0, chunks_per_tile, step=2)
    def _chunk(c):
      g0.wait()
      s0 = pltpu.async_copy(buf0_v, acc_sh.at[dst_v.at[c]], ssem0, add=True)
      g1 = pltpu.async_copy(tab_sh.at[src_v.at[c + 1]], buf1_v, gsem1)
      s0.wait()
      g1.wait()
      s1 = pltpu.async_copy(buf1_v, acc_sh.at[dst_v.at[c + 1]], ssem1,
                            add=True)

      @pl.when(c + 2 < chunks_per_tile)
      def _():
        pltpu.async_copy(tab_sh.at[src_v.at[c + 2]], buf0_v, gsem0)

      s1.wait()

    plsc.subcore_barrier()
    pltpu.sync_copy(acc_sh.at[pl.ds(sid * rows_per_sub, rows_per_sub)],
                    out_hbm.at[cid, pl.ds(sid * rows_per_sub, rows_per_sub)])

  return pl.kernel(
      body,
      out_type=jax.ShapeDtypeStruct((N_CORES, n_acc, d), dtype),
      mesh=mesh,
      compiler_params=pltpu.CompilerParams(use_tc_tiling_on_sc=False),
      scratch_types=[
          pltpu.VMEM((chunks_per_tile, CHUNK), jnp.int32),
          pltpu.VMEM((chunks_per_tile, CHUNK), jnp.int32),
          pltpu.VMEM((CHUNK, d), dtype),
          pltpu.VMEM((CHUNK, d), dtype),
          pltpu.VMEM_SHARED((n_rows, d), dtype),
          pltpu.VMEM_SHARED((n_acc, d), dtype),
          pltpu.SemaphoreType.DMA,
          pltpu.SemaphoreType.DMA,
          pltpu.SemaphoreType.DMA,
          pltpu.SemaphoreType.DMA,
      ],
  )


_DOT_FAST = functools.partial(jnp.dot, preferred_element_type=jnp.float32,
                              precision=lax.Precision.DEFAULT)
_DOT = functools.partial(jnp.dot, preferred_element_type=jnp.float32,
                         precision=lax.Precision.HIGHEST)


def _p_body(x_ref, wl_ref, pext_ref):
  x = x_ref[...]
  p = _DOT_FAST(x, wl_ref[...])
  ones = jnp.ones((x.shape[0], 32), jnp.float32)
  pext_ref[...] = jnp.concatenate([p, ones], axis=1).astype(jnp.bfloat16)


def _base_body(x_ref, wr_ref, b_ref, base_ref):
  base_ref[...] = _DOT(x_ref[...], wr_ref[...]) + b_ref[...]


def _mid_body(part_ref, base_ref, g_ref, bt_ref, wl_ref,
              q_ref, h_ref, cnt_ref):
  hid = base_ref.shape[1]
  agg = (part_ref[0, :, :hid].astype(jnp.float32) +
         part_ref[1, :, :hid].astype(jnp.float32))
  cnt = jnp.maximum(
      part_ref[0, :, hid:hid + 1].astype(jnp.float32) +
      part_ref[1, :, hid:hid + 1].astype(jnp.float32), 1.0)
  h = agg / cnt + base_ref[...]
  mu = jnp.mean(h, axis=0, keepdims=True)
  var = jnp.mean((h - mu) ** 2, axis=0, keepdims=True)
  h = (h - mu) / jnp.sqrt(var + 1e-5) * g_ref[...] + bt_ref[...]
  h = jnp.maximum(h, 0.0)
  q_ref[...] = _DOT_FAST(h, wl_ref[...]).astype(jnp.bfloat16)
  h_ref[...] = h
  cnt_ref[...] = cnt


def _s_body(h_ref, wr_ref, b2_ref, s_ref):
  s_ref[...] = _DOT(h_ref[...], wr_ref[...]) + b2_ref[...]


def _out_body(part_ref, s_ref, cnt_ref, out_ref):
  aggq = (part_ref[0].astype(jnp.float32) +
          part_ref[1].astype(jnp.float32))
  z = aggq / cnt_ref[...] + s_ref[...]
  norm = jnp.sqrt(jnp.sum(z * z, axis=1, keepdims=True))
  out_ref[...] = z / jnp.maximum(norm, 1e-12)


@jax.jit
def kernel(x, edge_index, W1l, b1l, W1r, b1r, gamma, beta, W2l, b2l, W2r, b2r):
  n, in_dim = x.shape
  hid = W1l.shape[1]
  out_dim = W2l.shape[1]
  n_edges = edge_index.shape[1]

  e_pad = -(-n_edges // (2 * N_TILES * CHUNK)) * (2 * N_TILES * CHUNK)
  n_acc = -(-(n + (1 if e_pad != n_edges else 0)) //
            (8 * N_SUBCORES)) * (8 * N_SUBCORES)

  src = edge_index[0].astype(jnp.int32)
  dst = edge_index[1].astype(jnp.int32)
  if e_pad != n_edges:
    src = jnp.concatenate([src, jnp.zeros((e_pad - n_edges,), jnp.int32)])
    dst = jnp.concatenate(
        [dst, jnp.full((e_pad - n_edges,), n_acc - 1, jnp.int32)])
  total_chunks = e_pad // CHUNK
  chunks_per_tile = total_chunks // N_TILES
  src2d = src.reshape(total_chunks, CHUNK)
  dst2d = dst.reshape(total_chunks, CHUNK)

  dp1 = hid + 32
  zeros1 = jnp.zeros((n_acc, dp1), jnp.bfloat16)
  zeros2 = jnp.zeros((n_acc, out_dim), jnp.bfloat16)

  blk = 1000
  grid = n // blk
  b1 = (b1l + b1r).reshape(1, hid)
  pext_pad = pl.pallas_call(
      _p_body,
      grid=(grid,),
      in_specs=[
          pl.BlockSpec((blk, in_dim), lambda i: (i, 0)),
          pl.BlockSpec((in_dim, hid), lambda i: (0, 0)),
      ],
      out_specs=pl.BlockSpec((blk, dp1), lambda i: (i, 0)),
      out_shape=jax.ShapeDtypeStruct((n, dp1), jnp.bfloat16),
  )(x, W1l)

  part1 = _make_seg_sum(n, n_acc, dp1, chunks_per_tile)(
      pext_pad, src2d, dst2d, zeros1)

  base = pl.pallas_call(
      _base_body,
      grid=(grid,),
      in_specs=[
          pl.BlockSpec((blk, in_dim), lambda i: (i, 0)),
          pl.BlockSpec((in_dim, hid), lambda i: (0, 0)),
          pl.BlockSpec((1, hid), lambda i: (0, 0)),
      ],
      out_specs=pl.BlockSpec((blk, hid), lambda i: (i, 0)),
      out_shape=jax.ShapeDtypeStruct((n, hid), jnp.float32),
  )(x, W1r, b1)

  b2 = (b2l + b2r).reshape(1, out_dim)
  q, h, cnt = pl.pallas_call(
      _mid_body,
      compiler_params=pltpu.CompilerParams(
          vmem_limit_bytes=100 * 1024 * 1024),
      grid=(1,),
      in_specs=[
          pl.BlockSpec((2, n, dp1), lambda i: (0, 0, 0)),
          pl.BlockSpec((n, hid), lambda i: (0, 0)),
          pl.BlockSpec((1, hid), lambda i: (0, 0)),
          pl.BlockSpec((1, hid), lambda i: (0, 0)),
          pl.BlockSpec((hid, out_dim), lambda i: (0, 0)),
      ],
      out_specs=[
          pl.BlockSpec((n, out_dim), lambda i: (0, 0)),
          pl.BlockSpec((n, hid), lambda i: (0, 0)),
          pl.BlockSpec((n, 1), lambda i: (0, 0)),
      ],
      out_shape=[
          jax.ShapeDtypeStruct((n, out_dim), jnp.bfloat16),
          jax.ShapeDtypeStruct((n, hid), jnp.float32),
          jax.ShapeDtypeStruct((n, 1), jnp.float32),
      ],
  )(part1, base, gamma.reshape(1, hid), beta.reshape(1, hid), W2l)

  part2 = _make_seg_sum(n, n_acc, out_dim, chunks_per_tile)(
      q, src2d, dst2d, zeros2)

  s = pl.pallas_call(
      _s_body,
      grid=(grid,),
      in_specs=[
          pl.BlockSpec((blk, hid), lambda i: (i, 0)),
          pl.BlockSpec((hid, out_dim), lambda i: (0, 0)),
          pl.BlockSpec((1, out_dim), lambda i: (0, 0)),
      ],
      out_specs=pl.BlockSpec((blk, out_dim), lambda i: (i, 0)),
      out_shape=jax.ShapeDtypeStruct((n, out_dim), jnp.float32),
  )(h, W2r, b2)

  z = pl.pallas_call(
      _out_body,
      grid=(grid,),
      in_specs=[
          pl.BlockSpec((2, blk, out_dim), lambda i: (0, i, 0)),
          pl.BlockSpec((blk, out_dim), lambda i: (i, 0)),
          pl.BlockSpec((blk, 1), lambda i: (i, 0)),
      ],
      out_specs=pl.BlockSpec((blk, out_dim), lambda i: (i, 0)),
      out_shape=jax.ShapeDtypeStruct((n, out_dim), jnp.float32),
  )(part2, s, cnt)
  return z


# speedup vs baseline: 1.5033x; 1.0149x over previous
import functools
import jax
import jax.numpy as jnp
from jax import lax
from jax.experimental import pallas as pl
from jax.experimental.pallas import tpu as pltpu
from jax.experimental.pallas import tpu_sc as plsc

N_CORES = 2
N_SUBCORES = 16
N_TILES = N_CORES * N_SUBCORES
CHUNK = 125


def _make_seg_sum(n_rows: int, n_acc: int, d: int, chunks_per_tile: int,
                  dtype=jnp.bfloat16):
  rows_per_sub = n_acc // N_SUBCORES
  tab_per_sub = n_rows // N_SUBCORES
  mesh = plsc.VectorSubcoreMesh(core_axis_name="c", subcore_axis_name="s")

  def body(rows_hbm, src_hbm, dst_hbm, zeros_hbm, out_hbm,
           src_v, dst_v, buf0_v, buf1_v, tab_sh, acc_sh,
           gsem0, gsem1, ssem0, ssem1):
    cid = lax.axis_index("c")
    sid = lax.axis_index("s")
    wid = cid * N_SUBCORES + sid
    pltpu.sync_copy(rows_hbm.at[pl.ds(sid * tab_per_sub, tab_per_sub)],
                    tab_sh.at[pl.ds(sid * tab_per_sub, tab_per_sub)])
    pltpu.sync_copy(zeros_hbm.at[pl.ds(sid * rows_per_sub, rows_per_sub)],
                    acc_sh.at[pl.ds(sid * rows_per_sub, rows_per_sub)])
    pltpu.sync_copy(src_hbm.at[pl.ds(wid * chunks_per_tile, chunks_per_tile)],
                    src_v)
    pltpu.sync_copy(dst_hbm.at[pl.ds(wid * chunks_per_tile, chunks_per_tile)],
                    dst_v)
    plsc.subcore_barrier()

    g0 = pltpu.async_copy(tab_sh.at[src_v.at[0]], buf0_v, gsem0)

    @pl.loop(0, chunks_per_tile, step=2)
    def _chunk(c):
      g0.wait()
      s0 = pltpu.async_copy(buf0_v, acc_sh.at[dst_v.at[c]], ssem0, add=True)
      g1 = pltpu.async_copy(tab_sh.at[src_v.at[c + 1]], buf1_v, gsem1)
      s0.wait()
      g1.wait()
      s1 = pltpu.async_copy(buf1_v, acc_sh.at[dst_v.at[c + 1]], ssem1,
                            add=True)

      @pl.when(c + 2 < chunks_per_tile)
      def _():
        pltpu.async_copy(tab_sh.at[src_v.at[c + 2]], buf0_v, gsem0)

      s1.wait()

    plsc.subcore_barrier()
    pltpu.sync_copy(acc_sh.at[pl.ds(sid * rows_per_sub, rows_per_sub)],
                    out_hbm.at[cid, pl.ds(sid * rows_per_sub, rows_per_sub)])

  return pl.kernel(
      body,
      out_type=jax.ShapeDtypeStruct((N_CORES, n_acc, d), dtype),
      mesh=mesh,
      compiler_params=pltpu.CompilerParams(use_tc_tiling_on_sc=False),
      scratch_types=[
          pltpu.VMEM((chunks_per_tile, CHUNK), jnp.int32),
          pltpu.VMEM((chunks_per_tile, CHUNK), jnp.int32),
          pltpu.VMEM((CHUNK, d), dtype),
          pltpu.VMEM((CHUNK, d), dtype),
          pltpu.VMEM_SHARED((n_rows, d), dtype),
          pltpu.VMEM_SHARED((n_acc, d), dtype),
          pltpu.SemaphoreType.DMA,
          pltpu.SemaphoreType.DMA,
          pltpu.SemaphoreType.DMA,
          pltpu.SemaphoreType.DMA,
      ],
  )


_DOT_FAST = functools.partial(jnp.dot, preferred_element_type=jnp.float32,
                              precision=lax.Precision.DEFAULT)
_DOT = functools.partial(jnp.dot, preferred_element_type=jnp.float32,
                         precision=lax.Precision.HIGHEST)


def _p_body(x_ref, wl_ref, pext_ref):
  x = x_ref[...]
  p = _DOT_FAST(x, wl_ref[...])
  ones = jnp.ones((x.shape[0], 32), jnp.float32)
  pext_ref[...] = jnp.concatenate([p, ones], axis=1).astype(jnp.bfloat16)


def _base_body(x_ref, wr_ref, b_ref, base_ref):
  base_ref[...] = _DOT(x_ref[...], wr_ref[...]) + b_ref[...]


def _mid_body(part_ref, base_ref, g_ref, bt_ref, wl_ref,
              q_ref, h_ref, cnt_ref):
  hid = base_ref.shape[1]
  agg = (part_ref[0, :, :hid].astype(jnp.float32) +
         part_ref[1, :, :hid].astype(jnp.float32))
  cnt = jnp.maximum(
      part_ref[0, :, hid:hid + 1].astype(jnp.float32) +
      part_ref[1, :, hid:hid + 1].astype(jnp.float32), 1.0)
  h = agg / cnt + base_ref[...]
  mu = jnp.mean(h, axis=0, keepdims=True)
  var = jnp.mean((h - mu) ** 2, axis=0, keepdims=True)
  h = (h - mu) / jnp.sqrt(var + 1e-5) * g_ref[...] + bt_ref[...]
  h = jnp.maximum(h, 0.0)
  q_ref[...] = _DOT_FAST(h, wl_ref[...]).astype(jnp.bfloat16)
  h_ref[...] = h
  cnt_ref[...] = cnt


def _s_body(h_ref, wr_ref, b2_ref, s_ref):
  s_ref[...] = _DOT(h_ref[...], wr_ref[...]) + b2_ref[...]


def _out_body(part_ref, s_ref, cnt_ref, out_ref):
  aggq = (part_ref[0].astype(jnp.float32) +
          part_ref[1].astype(jnp.float32))
  z = aggq / cnt_ref[...] + s_ref[...]
  norm = jnp.sqrt(jnp.sum(z * z, axis=1, keepdims=True))
  out_ref[...] = z / jnp.maximum(norm, 1e-12)


@jax.jit
def kernel(x, edge_index, W1l, b1l, W1r, b1r, gamma, beta, W2l, b2l, W2r, b2r):
  n, in_dim = x.shape
  hid = W1l.shape[1]
  out_dim = W2l.shape[1]
  n_edges = edge_index.shape[1]

  e_pad = -(-n_edges // (2 * N_TILES * CHUNK)) * (2 * N_TILES * CHUNK)
  n_acc = -(-(n + (1 if e_pad != n_edges else 0)) //
            (8 * N_SUBCORES)) * (8 * N_SUBCORES)

  src = edge_index[0].astype(jnp.int32)
  dst = edge_index[1].astype(jnp.int32)
  if e_pad != n_edges:
    src = jnp.concatenate([src, jnp.zeros((e_pad - n_edges,), jnp.int32)])
    dst = jnp.concatenate(
        [dst, jnp.full((e_pad - n_edges,), n_acc - 1, jnp.int32)])
  total_chunks = e_pad // CHUNK
  chunks_per_tile = total_chunks // N_TILES
  src2d = src.reshape(total_chunks, CHUNK)
  dst2d = dst.reshape(total_chunks, CHUNK)

  dp1 = hid + 32
  zeros1 = jnp.zeros((n_acc, dp1), jnp.bfloat16)
  zeros2 = jnp.zeros((n_acc, out_dim), jnp.bfloat16)

  blk = 2000
  grid = n // blk
  b1 = (b1l + b1r).reshape(1, hid)
  pext_pad = pl.pallas_call(
      _p_body,
      grid=(grid,),
      in_specs=[
          pl.BlockSpec((blk, in_dim), lambda i: (i, 0)),
          pl.BlockSpec((in_dim, hid), lambda i: (0, 0)),
      ],
      out_specs=pl.BlockSpec((blk, dp1), lambda i: (i, 0)),
      out_shape=jax.ShapeDtypeStruct((n, dp1), jnp.bfloat16),
  )(x, W1l)

  part1 = _make_seg_sum(n, n_acc, dp1, chunks_per_tile)(
      pext_pad, src2d, dst2d, zeros1)

  base = pl.pallas_call(
      _base_body,
      grid=(grid,),
      in_specs=[
          pl.BlockSpec((blk, in_dim), lambda i: (i, 0)),
          pl.BlockSpec((in_dim, hid), lambda i: (0, 0)),
          pl.BlockSpec((1, hid), lambda i: (0, 0)),
      ],
      out_specs=pl.BlockSpec((blk, hid), lambda i: (i, 0)),
      out_shape=jax.ShapeDtypeStruct((n, hid), jnp.float32),
  )(x, W1r, b1)

  b2 = (b2l + b2r).reshape(1, out_dim)
  q, h, cnt = pl.pallas_call(
      _mid_body,
      compiler_params=pltpu.CompilerParams(
          vmem_limit_bytes=100 * 1024 * 1024),
      grid=(1,),
      in_specs=[
          pl.BlockSpec((2, n, dp1), lambda i: (0, 0, 0)),
          pl.BlockSpec((n, hid), lambda i: (0, 0)),
          pl.BlockSpec((1, hid), lambda i: (0, 0)),
          pl.BlockSpec((1, hid), lambda i: (0, 0)),
          pl.BlockSpec((hid, out_dim), lambda i: (0, 0)),
      ],
      out_specs=[
          pl.BlockSpec((n, out_dim), lambda i: (0, 0)),
          pl.BlockSpec((n, hid), lambda i: (0, 0)),
          pl.BlockSpec((n, 1), lambda i: (0, 0)),
      ],
      out_shape=[
          jax.ShapeDtypeStruct((n, out_dim), jnp.bfloat16),
          jax.ShapeDtypeStruct((n, hid), jnp.float32),
          jax.ShapeDtypeStruct((n, 1), jnp.float32),
      ],
  )(part1, base, gamma.reshape(1, hid), beta.reshape(1, hid), W2l)

  part2 = _make_seg_sum(n, n_acc, out_dim, chunks_per_tile)(
      q, src2d, dst2d, zeros2)

  s = pl.pallas_call(
      _s_body,
      grid=(grid,),
      in_specs=[
          pl.BlockSpec((blk, hid), lambda i: (i, 0)),
          pl.BlockSpec((hid, out_dim), lambda i: (0, 0)),
          pl.BlockSpec((1, out_dim), lambda i: (0, 0)),
      ],
      out_specs=pl.BlockSpec((blk, out_dim), lambda i: (i, 0)),
      out_shape=jax.ShapeDtypeStruct((n, out_dim), jnp.float32),
  )(h, W2r, b2)

  z = pl.pallas_call(
      _out_body,
      grid=(grid,),
      in_specs=[
          pl.BlockSpec((2, blk, out_dim), lambda i: (0, i, 0)),
          pl.BlockSpec((blk, out_dim), lambda i: (i, 0)),
          pl.BlockSpec((blk, 1), lambda i: (i, 0)),
      ],
      out_specs=pl.BlockSpec((blk, out_dim), lambda i: (i, 0)),
      out_shape=jax.ShapeDtypeStruct((n, out_dim), jnp.float32),
  )(part2, s, cnt)
  return z
